# Initial kernel scaffold; baseline (speedup 1.0000x reference)
import jax
import jax.numpy as jnp
from jax.experimental import pallas as pl


def kernel(embedding, top_to_bottom_edge_index, W, b):
    raise NotImplementedError("write your pallas kernel here")


# speedup vs baseline: 12.9325x; 12.9325x over previous
import functools

import jax
import jax.numpy as jnp
from jax import lax
from jax.experimental import pallas as pl
from jax.experimental.pallas import tpu as pltpu
from jax.experimental.pallas import tpu_sc as plsc

N_NODES = 10000
N_EDGES = 320000
D = 128

NC = 2
NS = 16
NW = NC * NS

N_PAD = 10240
TRASH = N_PAD - 1
E_PAD = 327680
EB = E_PAD // NW
CHUNK = 128
NCHUNK = EB // CHUNK
ROWS_PER_TILE = N_PAD // NS


def _deg_body(dst_hbm, deg_out, dst_v, ones_v, zero_v, deg_sh):
    c = lax.axis_index("c")
    s = lax.axis_index("s")
    w = c * NS + s

    pltpu.sync_copy(dst_hbm.at[w], dst_v)

    z = jnp.zeros((16,), jnp.float32)
    for j in range(8):
        ones_v[pl.ds(j * 16, 16)] = z + 1.0
    for j in range(ROWS_PER_TILE // 16):
        zero_v[pl.ds(j * 16, 16)] = z
    pltpu.sync_copy(zero_v, deg_sh.at[pl.ds(s * ROWS_PER_TILE, ROWS_PER_TILE)])
    plsc.subcore_barrier()

    def chunk(i, carry):
        pltpu.sync_copy(ones_v, deg_sh.at[dst_v.at[i]], add=True)
        return carry

    lax.fori_loop(0, NCHUNK, chunk, 0)
    plsc.subcore_barrier()

    pltpu.sync_copy(
        deg_sh.at[pl.ds(s * ROWS_PER_TILE, ROWS_PER_TILE)],
        deg_out.at[c, pl.ds(s * ROWS_PER_TILE, ROWS_PER_TILE)],
    )


def _deg_partials(dst_r):
    mesh = plsc.VectorSubcoreMesh(
        core_axis_name="c", subcore_axis_name="s", num_cores=NC, num_subcores=NS
    )
    f = pl.kernel(
        _deg_body,
        out_type=jax.ShapeDtypeStruct((NC, N_PAD), jnp.float32),
        mesh=mesh,
        scratch_types=[
            pltpu.VMEM((NCHUNK, CHUNK), jnp.int32),
            pltpu.VMEM((CHUNK,), jnp.float32),
            pltpu.VMEM((ROWS_PER_TILE,), jnp.float32),
            pltpu.VMEM_SHARED((N_PAD,), jnp.float32),
        ],
    )
    return f(dst_r)


def _msg_body(h2_hbm, src_hbm, dst_hbm, acc_out, src_v, dst_v, rows_v, acc_sh, sem):
    c = lax.axis_index("c")
    s = lax.axis_index("s")
    w = c * NS + s

    pltpu.sync_copy(src_hbm.at[w], src_v)
    pltpu.sync_copy(dst_hbm.at[w], dst_v)

    z = jnp.zeros((16,), jnp.float32)

    def zrow(i, carry):
        for j in range(D // 16):
            rows_v[i, pl.ds(j * 16, 16)] = z
        return carry

    lax.fori_loop(0, CHUNK, zrow, 0)
    for r in range(ROWS_PER_TILE // CHUNK):
        pltpu.sync_copy(
            rows_v, acc_sh.at[pl.ds(s * ROWS_PER_TILE + r * CHUNK, CHUNK)]
        )
    plsc.subcore_barrier()

    def chunk(i, carry):
        pltpu.async_copy(h2_hbm.at[src_v.at[i]], rows_v, sem).wait()
        pltpu.sync_copy(rows_v, acc_sh.at[dst_v.at[i]], add=True)
        return carry

    lax.fori_loop(0, NCHUNK, chunk, 0)
    plsc.subcore_barrier()

    pltpu.sync_copy(
        acc_sh.at[pl.ds(s * ROWS_PER_TILE, ROWS_PER_TILE)],
        acc_out.at[c, pl.ds(s * ROWS_PER_TILE, ROWS_PER_TILE)],
    )


def _msg_partials(h2, src_r, dst_r):
    mesh = plsc.VectorSubcoreMesh(
        core_axis_name="c", subcore_axis_name="s", num_cores=NC, num_subcores=NS
    )
    f = pl.kernel(
        _msg_body,
        out_type=jax.ShapeDtypeStruct((NC, N_PAD, D), jnp.float32),
        mesh=mesh,
        scratch_types=[
            pltpu.VMEM((NCHUNK, CHUNK), jnp.int32),
            pltpu.VMEM((NCHUNK, CHUNK), jnp.int32),
            pltpu.VMEM((CHUNK, D), jnp.float32),
            pltpu.VMEM_SHARED((N_PAD, D), jnp.float32),
            pltpu.SemaphoreType.DMA,
        ],
    )
    return f(h2, src_r, dst_r)


def _h2_body(emb_ref, w_ref, deg_ref, out_ref):
    deg = deg_ref[0, :] + deg_ref[1, :] + 1.0
    dis = lax.rsqrt(deg)
    h = jnp.dot(emb_ref[...], w_ref[...], preferred_element_type=jnp.float32)
    out_ref[...] = h * dis[:, None]


def _h2(emb_pad, W, deg):
    bm = 512
    grid = N_PAD // bm
    return pl.pallas_call(
        _h2_body,
        grid=(grid,),
        in_specs=[
            pl.BlockSpec((bm, D), lambda i: (i, 0)),
            pl.BlockSpec((D, D), lambda i: (0, 0)),
            pl.BlockSpec((NC, bm), lambda i: (0, i)),
        ],
        out_specs=pl.BlockSpec((bm, D), lambda i: (i, 0)),
        out_shape=jax.ShapeDtypeStruct((N_PAD, D), jnp.float32),
    )(emb_pad, W, deg)


def _final_body(acc_ref, h2_ref, deg_ref, b_ref, out_ref):
    deg = deg_ref[0, :] + deg_ref[1, :] + 1.0
    dis = lax.rsqrt(deg)
    tot = acc_ref[0] + acc_ref[1] + h2_ref[...]
    out_ref[...] = tot * dis[:, None] + b_ref[...]


def _final(acc, h2, deg, b2d):
    bm = 512
    grid = N_PAD // bm
    return pl.pallas_call(
        _final_body,
        grid=(grid,),
        in_specs=[
            pl.BlockSpec((NC, bm, D), lambda i: (0, i, 0)),
            pl.BlockSpec((bm, D), lambda i: (i, 0)),
            pl.BlockSpec((NC, bm), lambda i: (0, i)),
            pl.BlockSpec((1, D), lambda i: (0, 0)),
        ],
        out_specs=pl.BlockSpec((bm, D), lambda i: (i, 0)),
        out_shape=jax.ShapeDtypeStruct((N_PAD, D), jnp.float32),
    )(acc, h2, deg, b2d)


@jax.jit
def kernel(embedding, top_to_bottom_edge_index, W, b):
    src = top_to_bottom_edge_index[0].astype(jnp.int32)
    dst = top_to_bottom_edge_index[1].astype(jnp.int32)
    pad = E_PAD - N_EDGES
    src_r = jnp.concatenate(
        [src, jnp.zeros((pad,), jnp.int32)]
    ).reshape(NW, NCHUNK, CHUNK)
    dst_r = jnp.concatenate(
        [dst, jnp.full((pad,), TRASH, jnp.int32)]
    ).reshape(NW, NCHUNK, CHUNK)
    emb_pad = jnp.pad(embedding, ((0, N_PAD - N_NODES), (0, 0)))

    deg = _deg_partials(dst_r)
    h2 = _h2(emb_pad, W, deg)
    acc = _msg_partials(h2, src_r, dst_r)
    out = _final(acc, h2, deg, b.reshape(1, D))
    return out[:N_NODES]


# speedup vs baseline: 39.3522x; 3.0429x over previous
import functools

import jax
import jax.numpy as jnp
from jax import lax
from jax.experimental import pallas as pl
from jax.experimental.pallas import tpu as pltpu
from jax.experimental.pallas import tpu_sc as plsc

N_NODES = 10000
N_EDGES = 320000
D = 128

NC = 2
NS = 16
NW = NC * NS

N_PAD = 10240
E_PAD = 327680
EB = E_PAD // NW
CHUNK = 128
NCHUNK = EB // CHUNK
HALF = NCHUNK // 2
ROWS_PER_TILE = N_PAD // NS
NBUF = 2


def _deg_body(dst_hbm, deg_out, dst_v, ones_v, zero_v, deg_sh):
    c = lax.axis_index("c")
    s = lax.axis_index("s")
    w = c * NS + s

    pltpu.sync_copy(dst_hbm.at[w], dst_v)

    z = jnp.zeros((16,), jnp.float32)
    for j in range(8):
        ones_v[pl.ds(j * 16, 16)] = z + 1.0
    for j in range(ROWS_PER_TILE // 16):
        zero_v[pl.ds(j * 16, 16)] = z
    pltpu.sync_copy(zero_v, deg_sh.at[pl.ds(s * ROWS_PER_TILE, ROWS_PER_TILE)])
    plsc.subcore_barrier()

    def chunk(i, carry):
        pltpu.sync_copy(ones_v, deg_sh.at[dst_v.at[i]], add=True)
        return carry

    lax.fori_loop(0, NCHUNK, chunk, 0)
    plsc.subcore_barrier()

    pltpu.sync_copy(
        deg_sh.at[pl.ds(s * ROWS_PER_TILE, ROWS_PER_TILE)],
        deg_out.at[c, pl.ds(s * ROWS_PER_TILE, ROWS_PER_TILE)],
    )


def _deg_partials(dst_r):
    mesh = plsc.VectorSubcoreMesh(
        core_axis_name="c", subcore_axis_name="s", num_cores=NC, num_subcores=NS
    )
    f = pl.kernel(
        _deg_body,
        out_type=jax.ShapeDtypeStruct((NC, N_PAD), jnp.float32),
        mesh=mesh,
        scratch_types=[
            pltpu.VMEM((NCHUNK, CHUNK), jnp.int32),
            pltpu.VMEM((CHUNK,), jnp.float32),
            pltpu.VMEM((ROWS_PER_TILE,), jnp.float32),
            pltpu.VMEM_SHARED((N_PAD,), jnp.float32),
        ],
    )
    return f(dst_r)


def _msg_body(h2_hbm, src_hbm, dst_hbm, acc_out,
              src_v, dst_v, rows0, rows1, acc_sh, sem):
    c = lax.axis_index("c")
    s = lax.axis_index("s")
    w = c * NS + s
    bufs = (rows0, rows1)

    pltpu.sync_copy(src_hbm.at[w], src_v)

    z = jnp.zeros((16,), jnp.float32)

    def zrow(i, carry):
        for j in range(D // 16):
            rows0[i, pl.ds(j * 16, 16)] = z
        return carry

    lax.fori_loop(0, CHUNK, zrow, 0)
    for r in range(ROWS_PER_TILE // CHUNK):
        pltpu.sync_copy(
            rows0, acc_sh.at[pl.ds(s * ROWS_PER_TILE + r * CHUNK, CHUNK)]
        )
    plsc.subcore_barrier()

    J = 8

    for h in range(2):
        pltpu.sync_copy(dst_hbm.at[w, pl.ds(h * HALF, HALF)], dst_v)
        hbase = h * HALF

        def outer(k, carry):
            base = hbase + k * J
            lbase = k * J
            d_prev = pltpu.async_copy(
                h2_hbm.at[src_v.at[base]], bufs[0], sem.at[0]
            )
            for j in range(J):
                cur = bufs[j % 2]
                if j + 1 < J:
                    d_next = pltpu.async_copy(
                        h2_hbm.at[src_v.at[base + j + 1]],
                        bufs[(j + 1) % 2],
                        sem.at[(j + 1) % 2],
                    )
                d_prev.wait()
                pltpu.sync_copy(cur, acc_sh.at[dst_v.at[lbase + j]], add=True)
                if j + 1 < J:
                    d_prev = d_next
            return carry

        lax.fori_loop(0, HALF // J, outer, 0)
    plsc.subcore_barrier()

    pltpu.sync_copy(
        acc_sh.at[pl.ds(s * ROWS_PER_TILE, ROWS_PER_TILE)],
        acc_out.at[c, pl.ds(s * ROWS_PER_TILE, ROWS_PER_TILE)],
    )


def _msg_partials(h2, src_m, dst_m):
    mesh = plsc.VectorSubcoreMesh(
        core_axis_name="c", subcore_axis_name="s", num_cores=NC, num_subcores=NS
    )
    f = pl.kernel(
        _msg_body,
        out_type=jax.ShapeDtypeStruct((NC, N_PAD, D), jnp.float32),
        mesh=mesh,
        scratch_types=[
            pltpu.VMEM((NCHUNK, CHUNK), jnp.int32),
            pltpu.VMEM((HALF, CHUNK), jnp.int32),
            pltpu.VMEM((CHUNK, D), jnp.float32),
            pltpu.VMEM((CHUNK, D), jnp.float32),
            pltpu.VMEM_SHARED((N_PAD, D), jnp.float32),
            pltpu.SemaphoreType.DMA((NBUF,)),
        ],
    )
    return f(h2, src_m, dst_m)


def _h2_body(emb_ref, w_ref, deg_ref, out_ref):
    deg = deg_ref[0, :] + deg_ref[1, :] + 1.0
    dis = lax.rsqrt(deg)
    h = jnp.dot(emb_ref[...], w_ref[...], preferred_element_type=jnp.float32)
    out_ref[...] = h * dis[:, None]


def _h2(emb_pad, W, deg):
    bm = 512
    grid = N_PAD // bm
    return pl.pallas_call(
        _h2_body,
        grid=(grid,),
        in_specs=[
            pl.BlockSpec((bm, D), lambda i: (i, 0)),
            pl.BlockSpec((D, D), lambda i: (0, 0)),
            pl.BlockSpec((NC, bm), lambda i: (0, i)),
        ],
        out_specs=pl.BlockSpec((bm, D), lambda i: (i, 0)),
        out_shape=jax.ShapeDtypeStruct((N_PAD, D), jnp.float32),
    )(emb_pad, W, deg)


def _final_body(acc_ref, h2_ref, deg_ref, b_ref, out_ref):
    deg = deg_ref[0, :] + deg_ref[1, :] + 1.0
    dis = lax.rsqrt(deg)
    tot = acc_ref[0] + acc_ref[1] + h2_ref[...]
    out_ref[...] = tot * dis[:, None] + b_ref[...]


def _final(acc, h2, deg, b2d):
    bm = 512
    grid = N_PAD // bm
    return pl.pallas_call(
        _final_body,
        grid=(grid,),
        in_specs=[
            pl.BlockSpec((NC, bm, D), lambda i: (0, i, 0)),
            pl.BlockSpec((bm, D), lambda i: (i, 0)),
            pl.BlockSpec((NC, bm), lambda i: (0, i)),
            pl.BlockSpec((1, D), lambda i: (0, 0)),
        ],
        out_specs=pl.BlockSpec((bm, D), lambda i: (i, 0)),
        out_shape=jax.ShapeDtypeStruct((N_PAD, D), jnp.float32),
    )(acc, h2, deg, b2d)


@jax.jit
def kernel(embedding, top_to_bottom_edge_index, W, b):
    src = top_to_bottom_edge_index[0].astype(jnp.int32)
    dst = top_to_bottom_edge_index[1].astype(jnp.int32)
    pad = E_PAD - N_EDGES
    pad_ar = jnp.arange(pad, dtype=jnp.int32)
    src_pad = pad_ar % N_NODES
    dst_pad = N_NODES + pad_ar % (N_PAD - N_NODES)
    src_m = jnp.concatenate([src, src_pad]).reshape(NW, NCHUNK, CHUNK)
    dst_m = jnp.concatenate([dst, dst_pad]).reshape(NW, NCHUNK, CHUNK)
    dst_r = dst_m
    emb_pad = jnp.pad(embedding, ((0, N_PAD - N_NODES), (0, 0)))

    deg = _deg_partials(dst_r)
    h2 = _h2(emb_pad, W, deg)
    acc = _msg_partials(h2, src_m, dst_m)
    out = _final(acc, h2, deg, b.reshape(1, D))
    return out[:N_NODES]
